# baseline (speedup 1.0000x reference)
import functools

import jax
import jax.numpy as jnp
from jax import lax
from jax.experimental import pallas as pl
from jax.experimental.pallas import tpu as pltpu
from jax.experimental.pallas import tpu_sc as plsc

N = 3976
E = 254464
D = 128
REL = 4
IDXN = 1024

NW = 32
NP = 4096
RPS = NP // 16
ZR = RPS // 8
DH = D // 2
CP = 56
EPS = E // 16
NCHP = EPS // CP
EPW = E // NW

_MESH = plsc.VectorSubcoreMesh(core_axis_name="c", subcore_axis_name="s")
_SC_PARAMS = pltpu.CompilerParams(needs_layout_passes=False,
                                  use_tc_tiling_on_sc=False)


@functools.partial(
    pl.kernel,
    mesh=_MESH,
    compiler_params=_SC_PARAMS,
    out_type=jax.ShapeDtypeStruct((NW, NP), jnp.float32),
    scratch_types=[
        pltpu.VMEM((NP,), jnp.float32),
        pltpu.VMEM((EPW,), jnp.int32),
    ],
)
def _deg_kernel(dst_hbm, out, deg_v, dstb):
    w = lax.axis_index("s") * 2 + lax.axis_index("c")
    zero = jnp.zeros((16,), jnp.float32)

    def zbody(i, c):
        deg_v[pl.ds(i * 16, 16)] = zero
        return c

    lax.fori_loop(0, NP // 16, zbody, 0)
    pltpu.sync_copy(dst_hbm.at[pl.ds(w * EPW, EPW)], dstb)
    ones = jnp.full((16,), 1.0, jnp.float32)

    def body(v, c):
        dv = dstb[pl.ds(v * 16, 16)]
        plsc.addupdate_scatter(deg_v, [dv], ones)
        return c

    lax.fori_loop(0, EPW // 16, body, 0)
    pltpu.sync_copy(deg_v, out.at[w])


@functools.partial(
    pl.kernel,
    mesh=_MESH,
    compiler_params=_SC_PARAMS,
    out_type=[
        jax.ShapeDtypeStruct((2, NP, 2 * DH), jnp.float32),
        jax.ShapeDtypeStruct((2, NP, DH), jnp.float32),
    ],
    scratch_types=[
        pltpu.VMEM_SHARED((NP, 2 * DH), jnp.float32),
        pltpu.VMEM_SHARED((NP, DH), jnp.float32),
    ] + [pltpu.VMEM((3, CP), jnp.int32) for _ in range(4)]
      + [pltpu.VMEM((CP, DH), jnp.float32) for _ in range(4)]
      + [pltpu.VMEM((CP, 2 * DH), jnp.float32) for _ in range(4)]
      + [pltpu.VMEM((CP, DH), jnp.float32) for _ in range(4)]
      + [
        pltpu.VMEM((ZR, 2 * DH), jnp.float32),
        pltpu.VMEM((ZR, DH), jnp.float32),
    ] + [pltpu.SemaphoreType.DMA for _ in range(12)],
)
def _edge_kernel(t_hbm, y_hbm, eidx_hbm, sn_out, g_out,
                 sn_sh, g_sh, i0, i1, i2, i3, t0, t1, t2, t3,
                 c0, c1, c2, c3, y0, y1, y2, y3, zb2, zb1,
                 sT0, sT1, sT2, sT3, sY0, sY1, sY2, sY3,
                 sS0, sS1, sS2, sS3):
    cid = lax.axis_index("c")
    sid = lax.axis_index("s")
    zero = jnp.zeros((16,), jnp.float32)

    def zb(i, c):
        for f in range(2 * DH // 16):
            zb2[i, pl.ds(f * 16, 16)] = zero
        for f in range(DH // 16):
            zb1[i, pl.ds(f * 16, 16)] = zero
        return c

    lax.fori_loop(0, ZR, zb, 0)
    rbase = sid * RPS
    for j in range(RPS // ZR):
        pltpu.sync_copy(zb2, sn_sh.at[pl.ds(rbase + j * ZR, ZR)])
        pltpu.sync_copy(zb1, g_sh.at[pl.ds(rbase + j * ZR, ZR)])
    plsc.subcore_barrier()

    t_half = t_hbm.at[cid]
    y_half = y_hbm.at[cid]
    slots = ((i0, t0, c0, y0, sT0, sY0, sS0),
             (i1, t1, c1, y1, sT1, sY1, sS1),
             (i2, t2, c2, y2, sT2, sY2, sS2),
             (i3, t3, c3, y3, sT3, sY3, sS3))

    def issue(ci, sl):
        idxb, rT, _, rY, sT, sY, _ = slots[sl]
        pltpu.sync_copy(eidx_hbm.at[sid, ci], idxb)
        pltpu.async_copy(t_half.at[idxb.at[1]], rT, sT)
        pltpu.async_copy(y_half.at[idxb.at[2]], rY, sY)

    def compute_and_fire(sl):
        idxb, rT, cb, rY, sT, sY, sS = slots[sl]
        pltpu.make_async_copy(t_half.at[idxb.at[1]], rT, sT).wait()

        def crow(r, c2):
            for f in range(DH // 16):
                t = rT[r, pl.ds(f * 16, 16)]
                e = jnp.exp(t)
                cb[r, pl.ds(f * 16, 16)] = e
                cb[r, pl.ds(DH + f * 16, 16)] = t * e
            return c2

        lax.fori_loop(0, CP, crow, 0)
        pltpu.make_async_copy(y_half.at[idxb.at[2]], rY, sY).wait()
        pltpu.async_copy(cb, sn_sh.at[idxb.at[0]], sS, add=True)
        pltpu.async_copy(rY, g_sh.at[idxb.at[0]], sS, add=True)

    def drain(sl):
        idxb, rT, cb, rY, sT, sY, sS = slots[sl]
        pltpu.make_async_copy(cb, sn_sh.at[idxb.at[0]], sS).wait()
        pltpu.make_async_copy(rY, g_sh.at[idxb.at[0]], sS).wait()

    issue(0, 0)
    issue(1, 1)

    def body(j, c):
        base = j * 4
        for k in range(4):
            ci = base + k
            compute_and_fire(k)

            @pl.when(ci >= 2)
            def _():
                drain((k - 2) % 4)

            @pl.when(ci + 2 < NCHP)
            def _():
                issue(ci + 2, (k + 2) % 4)
        return c

    lax.fori_loop(0, NCHP // 4, body, 0)
    drain(2)
    drain(3)
    plsc.subcore_barrier()

    pltpu.sync_copy(sn_sh.at[pl.ds(rbase, RPS)], sn_out.at[cid, pl.ds(rbase, RPS)])
    pltpu.sync_copy(g_sh.at[pl.ds(rbase, RPS)], g_out.at[cid, pl.ds(rbase, RPS)])


_ROWS = IDXN // NW


@functools.partial(
    pl.kernel,
    mesh=_MESH,
    compiler_params=_SC_PARAMS,
    out_type=jax.ShapeDtypeStruct((IDXN, D), jnp.float32),
    scratch_types=[
        pltpu.VMEM((_ROWS,), jnp.int32),
        pltpu.VMEM((_ROWS, D), jnp.float32),
        pltpu.SemaphoreType.DMA,
    ],
)
def _gather_kernel(h_hbm, idx_hbm, out, idx_v, rows_v, sem):
    w = lax.axis_index("s") * 2 + lax.axis_index("c")
    base = w * _ROWS
    pltpu.sync_copy(idx_hbm.at[pl.ds(base, _ROWS)], idx_v)
    pltpu.async_copy(h_hbm.at[idx_v], rows_v, sem).wait()
    pltpu.sync_copy(rows_v, out.at[pl.ds(base, _ROWS)])


def _ln(t, g_row, b_row):
    mu = t.mean(1, keepdims=True)
    var = ((t - mu) ** 2).mean(1, keepdims=True)
    return (t - mu) * jax.lax.rsqrt(var + 1e-5) * g_row + b_row


def _dinv_col(deg_ref):
    deg = jnp.sum(deg_ref[...], axis=0)[:N]
    return jnp.where(deg > 0, jax.lax.rsqrt(jnp.maximum(deg, 1.0)), 0.0)[:, None]


def _dense_stage(x, dinv_col, w_rel, pg_w, pg_b, ng_g, ng_b, tab_o, y_o):
    g = _ln(jax.nn.relu(x @ pg_w + pg_b), ng_g, ng_b)
    y_o[...] = x * dinv_col
    ps = [jnp.dot(x, w_rel[r], preferred_element_type=jnp.float32)
          for r in range(REL)]
    m = ps[0].max(0, keepdims=True)
    for r in range(1, REL):
        m = jnp.maximum(m, ps[r].max(0, keepdims=True))
    for r in range(REL):
        tab_o[:, r, :] = ps[r] - m
    return g, m


def _pre_body(x0_ref, deg_ref, proj_w, proj_b, bn_g, bn_b,
              wi_w, wi_b, pg_w, pg_b, ng_g, ng_b, w_rel,
              g_o, y_o, tab_o, m_o):
    dinv_col = _dinv_col(deg_ref)
    hp = x0_ref[...] @ proj_w[...] + proj_b[...]
    mu = hp.mean(0, keepdims=True)
    var = ((hp - mu) ** 2).mean(0, keepdims=True)
    h = jax.nn.relu((hp - mu) * jax.lax.rsqrt(var + 1e-5) * bn_g[...] + bn_b[...])
    x = h @ wi_w[...] + wi_b[...]
    g, m = _dense_stage(x, dinv_col, w_rel[...], pg_w[...], pg_b[...],
                        ng_g[...], ng_b[...], tab_o, y_o)
    g_o[...] = g
    m_o[...] = m


def _combine(g, sn_p, g2_p, m, dinv_col, co_w, co_b, n_g, n_b):
    s = jnp.concatenate([sn_p[0, :N, :DH], sn_p[1, :N, :DH]], axis=1)
    num = jnp.concatenate([sn_p[0, :N, DH:], sn_p[1, :N, DH:]], axis=1)
    g2 = jnp.concatenate([g2_p[0, :N, :], g2_p[1, :N, :]], axis=1)
    msg = jnp.where(s > 0, num / jnp.maximum(s, 1e-37) + m, 0.0)
    tot = g + g2 * dinv_col + 0.1 * jax.nn.relu(msg)
    return _ln(tot @ co_w + co_b, n_g, n_b)


def _mid_body(g_ref, sn_ref, g2_ref, m_ref, deg_ref,
              co_w, co_b, n_g, n_b,
              wi_w, wi_b, pg_w, pg_b, ng_g, ng_b, w_rel,
              g_o, y_o, tab_o, m_o):
    dinv_col = _dinv_col(deg_ref)
    h = _combine(g_ref[...], sn_ref[...], g2_ref[...], m_ref[...], dinv_col,
                 co_w[...], co_b[...], n_g[...], n_b[...])
    x = h @ wi_w[...] + wi_b[...]
    g, m = _dense_stage(x, dinv_col, w_rel[...], pg_w[...], pg_b[...],
                        ng_g[...], ng_b[...], tab_o, y_o)
    g_o[...] = g
    m_o[...] = m


def _post_body(g_ref, sn_ref, g2_ref, m_ref, deg_ref,
               co_w, co_b, n_g, n_b, h_o):
    dinv_col = _dinv_col(deg_ref)
    h = _combine(g_ref[...], sn_ref[...], g2_ref[...], m_ref[...], dinv_col,
                 co_w[...], co_b[...], n_g[...], n_b[...])
    h_o[...] = h * 0.5 * (1.0 + jax.lax.erf(h * (2.0 ** -0.5)))


_ND = jax.ShapeDtypeStruct((N, D), jnp.float32)
_DENSE_OUT = [_ND, _ND,
              jax.ShapeDtypeStruct((N, REL, D), jnp.float32),
              jax.ShapeDtypeStruct((1, D), jnp.float32)]

_TC_PARAMS = pltpu.CompilerParams(vmem_limit_bytes=100 * 1024 * 1024)
_pre_call = pl.pallas_call(_pre_body, out_shape=_DENSE_OUT, compiler_params=_TC_PARAMS)
_mid_call = pl.pallas_call(_mid_body, out_shape=_DENSE_OUT, compiler_params=_TC_PARAMS)
_post_call = pl.pallas_call(_post_body, out_shape=_ND, compiler_params=_TC_PARAMS)


def kernel(x, edge_index, idx, edge_type, params):
    src = edge_index[0]
    dst = edge_index[1]
    key2 = dst * 4 + edge_type
    eidx = jnp.stack([src, key2, dst]).reshape(3, 16, NCHP, CP)
    eidx = eidx.transpose(1, 2, 0, 3)

    deg_part = _deg_kernel(dst)

    def halves(a):
        return jnp.stack([a[:, :DH], a[:, DH:]])

    l1, l2 = params["layers"]

    def row(v):
        return v.reshape(1, D)

    g1, y1, tab1, m1 = _pre_call(
        x, deg_part, params["proj_w"], row(params["proj_b"]),
        row(params["bn_g"]), row(params["bn_b"]),
        l1["wi_w"], row(l1["wi_b"]), l1["pg_w"], row(l1["pg_b"]),
        row(l1["ng_g"]), row(l1["ng_b"]), l1["w_rel"])

    sn1, g21 = _edge_kernel(halves(tab1.reshape(N * REL, D)), halves(y1), eidx)

    g2_, y2, tab2, m2 = _mid_call(
        g1, sn1, g21, m1, deg_part,
        l1["co_w"], row(l1["co_b"]), row(l1["n_g"]), row(l1["n_b"]),
        l2["wi_w"], row(l2["wi_b"]), l2["pg_w"], row(l2["pg_b"]),
        row(l2["ng_g"]), row(l2["ng_b"]), l2["w_rel"])

    sn2, g22 = _edge_kernel(halves(tab2.reshape(N * REL, D)), halves(y2), eidx)

    h = _post_call(g2_, sn2, g22, m2, deg_part,
                   l2["co_w"], row(l2["co_b"]), row(l2["n_g"]), row(l2["n_b"]))

    return _gather_kernel(h, idx)


# speedup vs baseline: 2.2243x; 2.2243x over previous
import functools

import jax
import jax.numpy as jnp
from jax import lax
from jax.experimental import pallas as pl
from jax.experimental.pallas import tpu as pltpu
from jax.experimental.pallas import tpu_sc as plsc

N = 3976
E = 254464
D = 128
REL = 4
IDXN = 1024

NW = 32
NP = 4096
RPS = NP // 16
ZR = RPS // 8
DH = D // 2
CP = 56
EPS = E // 16
NCHP = EPS // CP
EPW = E // NW

_MESH = plsc.VectorSubcoreMesh(core_axis_name="c", subcore_axis_name="s")
_SC_PARAMS = pltpu.CompilerParams(needs_layout_passes=False,
                                  use_tc_tiling_on_sc=False)


@functools.partial(
    pl.kernel,
    mesh=_MESH,
    compiler_params=_SC_PARAMS,
    out_type=jax.ShapeDtypeStruct((NW, NP), jnp.float32),
    scratch_types=[
        pltpu.VMEM((NP,), jnp.float32),
        pltpu.VMEM((EPW,), jnp.int32),
    ],
)
def _deg_kernel(dst_hbm, out, deg_v, dstb):
    w = lax.axis_index("s") * 2 + lax.axis_index("c")
    zero = jnp.zeros((16,), jnp.float32)

    def zbody(i, c):
        deg_v[pl.ds(i * 16, 16)] = zero
        return c

    lax.fori_loop(0, NP // 16, zbody, 0)
    pltpu.sync_copy(dst_hbm.at[pl.ds(w * EPW, EPW)], dstb)
    ones = jnp.full((16,), 1.0, jnp.float32)

    def body(v, c):
        dv = dstb[pl.ds(v * 16, 16)]
        plsc.addupdate_scatter(deg_v, [dv], ones)
        return c

    lax.fori_loop(0, EPW // 16, body, 0)
    pltpu.sync_copy(deg_v, out.at[w])


@functools.partial(
    pl.kernel,
    mesh=_MESH,
    compiler_params=_SC_PARAMS,
    out_type=[
        jax.ShapeDtypeStruct((2, NP, DH), jnp.float32),
        jax.ShapeDtypeStruct((2, NP, DH), jnp.float32),
        jax.ShapeDtypeStruct((2, NP, DH), jnp.float32),
    ],
    scratch_types=[
        pltpu.VMEM_SHARED((NP, DH), jnp.float32),
        pltpu.VMEM_SHARED((NP, DH), jnp.float32),
        pltpu.VMEM_SHARED((NP, DH), jnp.float32),
    ] + [pltpu.VMEM((3, CP), jnp.int32) for _ in range(4)]
      + [pltpu.VMEM((CP, DH), jnp.float32) for _ in range(4)]
      + [pltpu.VMEM((CP, DH), jnp.float32) for _ in range(4)]
      + [pltpu.VMEM((CP, DH), jnp.float32) for _ in range(4)]
      + [
        pltpu.VMEM((ZR, DH), jnp.float32),
    ] + [pltpu.SemaphoreType.DMA for _ in range(12)],
)
def _edge_kernel(t_hbm, y_hbm, eidx_hbm, s_out, num_out, g_out,
                 s_sh, num_sh, g_sh, i0, i1, i2, i3, t0, t1, t2, t3,
                 c0, c1, c2, c3, y0, y1, y2, y3, zb1,
                 sT0, sT1, sT2, sT3, sY0, sY1, sY2, sY3,
                 sS0, sS1, sS2, sS3):
    cid = lax.axis_index("c")
    sid = lax.axis_index("s")
    zero = jnp.zeros((16,), jnp.float32)

    def zb(i, c):
        for f in range(DH // 16):
            zb1[i, pl.ds(f * 16, 16)] = zero
        return c

    lax.fori_loop(0, ZR, zb, 0)
    rbase = sid * RPS
    for j in range(RPS // ZR):
        pltpu.sync_copy(zb1, s_sh.at[pl.ds(rbase + j * ZR, ZR)])
        pltpu.sync_copy(zb1, num_sh.at[pl.ds(rbase + j * ZR, ZR)])
        pltpu.sync_copy(zb1, g_sh.at[pl.ds(rbase + j * ZR, ZR)])
    plsc.subcore_barrier()

    t_half = t_hbm.at[cid]
    y_half = y_hbm.at[cid]
    slots = ((i0, t0, c0, y0, sT0, sY0, sS0),
             (i1, t1, c1, y1, sT1, sY1, sS1),
             (i2, t2, c2, y2, sT2, sY2, sS2),
             (i3, t3, c3, y3, sT3, sY3, sS3))

    def issue(ci, sl):
        idxb, rT, _, rY, sT, sY, _ = slots[sl]
        pltpu.sync_copy(eidx_hbm.at[sid, ci], idxb)
        pltpu.async_copy(t_half.at[idxb.at[1]], rT, sT)
        pltpu.async_copy(y_half.at[idxb.at[2]], rY, sY)

    def compute_and_fire(sl):
        idxb, rT, cb, rY, sT, sY, sS = slots[sl]
        pltpu.make_async_copy(t_half.at[idxb.at[1]], rT, sT).wait()

        def crow(r, c2):
            for f in range(DH // 16):
                t = rT[r, pl.ds(f * 16, 16)]
                e = jnp.exp(t)
                cb[r, pl.ds(f * 16, 16)] = e
                rT[r, pl.ds(f * 16, 16)] = t * e
            return c2

        lax.fori_loop(0, CP, crow, 0)
        pltpu.make_async_copy(y_half.at[idxb.at[2]], rY, sY).wait()
        pltpu.async_copy(cb, s_sh.at[idxb.at[0]], sS, add=True)
        pltpu.async_copy(rT, num_sh.at[idxb.at[0]], sS, add=True)
        pltpu.async_copy(rY, g_sh.at[idxb.at[0]], sS, add=True)

    def drain(sl):
        idxb, rT, cb, rY, sT, sY, sS = slots[sl]
        pltpu.make_async_copy(cb, s_sh.at[idxb.at[0]], sS).wait()
        pltpu.make_async_copy(rT, num_sh.at[idxb.at[0]], sS).wait()
        pltpu.make_async_copy(rY, g_sh.at[idxb.at[0]], sS).wait()

    issue(0, 0)
    issue(1, 1)

    def body(j, c):
        base = j * 4
        for k in range(4):
            ci = base + k
            compute_and_fire(k)

            @pl.when(ci >= 2)
            def _():
                drain((k - 2) % 4)

            @pl.when(ci + 2 < NCHP)
            def _():
                issue(ci + 2, (k + 2) % 4)
        return c

    lax.fori_loop(0, NCHP // 4, body, 0)
    drain(2)
    drain(3)
    plsc.subcore_barrier()

    pltpu.sync_copy(s_sh.at[pl.ds(rbase, RPS)], s_out.at[cid, pl.ds(rbase, RPS)])
    pltpu.sync_copy(num_sh.at[pl.ds(rbase, RPS)], num_out.at[cid, pl.ds(rbase, RPS)])
    pltpu.sync_copy(g_sh.at[pl.ds(rbase, RPS)], g_out.at[cid, pl.ds(rbase, RPS)])


_ROWS = IDXN // NW


@functools.partial(
    pl.kernel,
    mesh=_MESH,
    compiler_params=_SC_PARAMS,
    out_type=jax.ShapeDtypeStruct((IDXN, D), jnp.float32),
    scratch_types=[
        pltpu.VMEM((_ROWS,), jnp.int32),
        pltpu.VMEM((_ROWS, D), jnp.float32),
        pltpu.SemaphoreType.DMA,
    ],
)
def _gather_kernel(h_hbm, idx_hbm, out, idx_v, rows_v, sem):
    w = lax.axis_index("s") * 2 + lax.axis_index("c")
    base = w * _ROWS
    pltpu.sync_copy(idx_hbm.at[pl.ds(base, _ROWS)], idx_v)
    pltpu.async_copy(h_hbm.at[idx_v], rows_v, sem).wait()
    pltpu.sync_copy(rows_v, out.at[pl.ds(base, _ROWS)])


def _ln(t, g_row, b_row):
    mu = t.mean(1, keepdims=True)
    var = ((t - mu) ** 2).mean(1, keepdims=True)
    return (t - mu) * jax.lax.rsqrt(var + 1e-5) * g_row + b_row


def _dinv_col(deg_ref):
    deg = jnp.sum(deg_ref[...], axis=0)[:N]
    return jnp.where(deg > 0, jax.lax.rsqrt(jnp.maximum(deg, 1.0)), 0.0)[:, None]


def _dense_stage(x, dinv_col, w_rel, pg_w, pg_b, ng_g, ng_b, tab_o, y_o):
    g = _ln(jax.nn.relu(x @ pg_w + pg_b), ng_g, ng_b)
    y_o[...] = x * dinv_col
    ps = [jnp.dot(x, w_rel[r], preferred_element_type=jnp.float32)
          for r in range(REL)]
    m = ps[0].max(0, keepdims=True)
    for r in range(1, REL):
        m = jnp.maximum(m, ps[r].max(0, keepdims=True))
    for r in range(REL):
        tab_o[:, r, :] = ps[r] - m
    return g, m


def _pre_body(x0_ref, deg_ref, proj_w, proj_b, bn_g, bn_b,
              wi_w, wi_b, pg_w, pg_b, ng_g, ng_b, w_rel,
              g_o, y_o, tab_o, m_o):
    dinv_col = _dinv_col(deg_ref)
    hp = x0_ref[...] @ proj_w[...] + proj_b[...]
    mu = hp.mean(0, keepdims=True)
    var = ((hp - mu) ** 2).mean(0, keepdims=True)
    h = jax.nn.relu((hp - mu) * jax.lax.rsqrt(var + 1e-5) * bn_g[...] + bn_b[...])
    x = h @ wi_w[...] + wi_b[...]
    g, m = _dense_stage(x, dinv_col, w_rel[...], pg_w[...], pg_b[...],
                        ng_g[...], ng_b[...], tab_o, y_o)
    g_o[...] = g
    m_o[...] = m


def _combine(g, s_p, num_p, g2_p, m, dinv_col, co_w, co_b, n_g, n_b):
    s = jnp.concatenate([s_p[0, :N, :], s_p[1, :N, :]], axis=1)
    num = jnp.concatenate([num_p[0, :N, :], num_p[1, :N, :]], axis=1)
    g2 = jnp.concatenate([g2_p[0, :N, :], g2_p[1, :N, :]], axis=1)
    msg = jnp.where(s > 0, num / jnp.maximum(s, 1e-37) + m, 0.0)
    tot = g + g2 * dinv_col + 0.1 * jax.nn.relu(msg)
    return _ln(tot @ co_w + co_b, n_g, n_b)


def _mid_body(g_ref, s_ref, num_ref, g2_ref, m_ref, deg_ref,
              co_w, co_b, n_g, n_b,
              wi_w, wi_b, pg_w, pg_b, ng_g, ng_b, w_rel,
              g_o, y_o, tab_o, m_o):
    dinv_col = _dinv_col(deg_ref)
    h = _combine(g_ref[...], s_ref[...], num_ref[...], g2_ref[...], m_ref[...],
                 dinv_col, co_w[...], co_b[...], n_g[...], n_b[...])
    x = h @ wi_w[...] + wi_b[...]
    g, m = _dense_stage(x, dinv_col, w_rel[...], pg_w[...], pg_b[...],
                        ng_g[...], ng_b[...], tab_o, y_o)
    g_o[...] = g
    m_o[...] = m


def _post_body(g_ref, s_ref, num_ref, g2_ref, m_ref, deg_ref,
               co_w, co_b, n_g, n_b, h_o):
    dinv_col = _dinv_col(deg_ref)
    h = _combine(g_ref[...], s_ref[...], num_ref[...], g2_ref[...], m_ref[...],
                 dinv_col, co_w[...], co_b[...], n_g[...], n_b[...])
    h_o[...] = h * 0.5 * (1.0 + jax.lax.erf(h * (2.0 ** -0.5)))


_ND = jax.ShapeDtypeStruct((N, D), jnp.float32)
_DENSE_OUT = [_ND, _ND,
              jax.ShapeDtypeStruct((N, REL, D), jnp.float32),
              jax.ShapeDtypeStruct((1, D), jnp.float32)]

_TC_PARAMS = pltpu.CompilerParams(vmem_limit_bytes=100 * 1024 * 1024)
_pre_call = pl.pallas_call(_pre_body, out_shape=_DENSE_OUT, compiler_params=_TC_PARAMS)
_mid_call = pl.pallas_call(_mid_body, out_shape=_DENSE_OUT, compiler_params=_TC_PARAMS)
_post_call = pl.pallas_call(_post_body, out_shape=_ND, compiler_params=_TC_PARAMS)


def kernel(x, edge_index, idx, edge_type, params):
    src = edge_index[0]
    dst = edge_index[1]
    key2 = dst * 4 + edge_type
    eidx = jnp.stack([src, key2, dst]).reshape(3, 16, NCHP, CP)
    eidx = eidx.transpose(1, 2, 0, 3)

    deg_part = _deg_kernel(dst)

    def halves(a):
        return jnp.stack([a[:, :DH], a[:, DH:]])

    l1, l2 = params["layers"]

    def row(v):
        return v.reshape(1, D)

    g1, y1, tab1, m1 = _pre_call(
        x, deg_part, params["proj_w"], row(params["proj_b"]),
        row(params["bn_g"]), row(params["bn_b"]),
        l1["wi_w"], row(l1["wi_b"]), l1["pg_w"], row(l1["pg_b"]),
        row(l1["ng_g"]), row(l1["ng_b"]), l1["w_rel"])

    s1, num1, g21 = _edge_kernel(halves(tab1.reshape(N * REL, D)), halves(y1), eidx)

    g2_, y2, tab2, m2 = _mid_call(
        g1, s1, num1, g21, m1, deg_part,
        l1["co_w"], row(l1["co_b"]), row(l1["n_g"]), row(l1["n_b"]),
        l2["wi_w"], row(l2["wi_b"]), l2["pg_w"], row(l2["pg_b"]),
        row(l2["ng_g"]), row(l2["ng_b"]), l2["w_rel"])

    s2, num2, g22 = _edge_kernel(halves(tab2.reshape(N * REL, D)), halves(y2), eidx)

    h = _post_call(g2_, s2, num2, g22, m2, deg_part,
                   l2["co_w"], row(l2["co_b"]), row(l2["n_g"]), row(l2["n_b"]))

    return _gather_kernel(h, idx)
